# initial kernel scaffold (unmeasured)
import jax
import jax.numpy as jnp
from jax import lax
from jax.experimental import pallas as pl
from jax.experimental.pallas import tpu as pltpu

N = 1024
D = 512


def kernel(x, dest):
    order = jnp.argsort(dest)
    xs = jnp.take(x, order, axis=0)
    z = jnp.sum(dest == 0).astype(jnp.int32).reshape(1)

    def body(z_ref, x_ref, out_ref, buf_ref, send_sem, recv_sem):
        my_x = lax.axis_index("x")
        my_y = lax.axis_index("y")
        z = z_ref[0]

        buf_ref[pl.ds(N, N), :] = x_ref[...]

        barrier = pltpu.get_barrier_semaphore()
        pl.semaphore_signal(
            barrier,
            inc=1,
            device_id=(1 - my_x, my_y),
            device_id_type=pl.DeviceIdType.MESH,
        )
        pl.semaphore_wait(barrier, 1)

        dst_start = (my_x + 1) * N - z
        rdma = pltpu.make_async_remote_copy(
            src_ref=x_ref,
            dst_ref=buf_ref.at[pl.ds(dst_start, N)],
            send_sem=send_sem,
            recv_sem=recv_sem,
            device_id=(1 - my_x, my_y),
            device_id_type=pl.DeviceIdType.MESH,
        )
        rdma.start()
        rdma.wait()

        out_ref[...] = buf_ref[pl.ds(N, N), :]

    return pl.pallas_call(
        body,
        out_shape=jax.ShapeDtypeStruct((N, D), jnp.float32),
        in_specs=[
            pl.BlockSpec(memory_space=pltpu.SMEM),
            pl.BlockSpec(memory_space=pltpu.VMEM),
        ],
        out_specs=pl.BlockSpec(memory_space=pltpu.VMEM),
        scratch_shapes=[
            pltpu.VMEM((3 * N, D), jnp.float32),
            pltpu.SemaphoreType.DMA,
            pltpu.SemaphoreType.DMA,
        ],
        compiler_params=pltpu.CompilerParams(collective_id=0),
    )(z, xs)


# baseline (device time: 39954 ns/iter reference)
import jax
import jax.numpy as jnp
from jax import lax
from jax.experimental import pallas as pl
from jax.experimental.pallas import tpu as pltpu

N = 1024
D = 512


def kernel(x, dest):
    order = jnp.argsort(dest)
    xs = jnp.take(x, order, axis=0)
    z = jnp.sum(dest == 0).astype(jnp.int32).reshape(1)

    def body(z_ref, x_ref, out_ref, pbuf_ref, send_sem, recv_sem):
        my_x = lax.axis_index("x")
        my_y = lax.axis_index("y")
        z = z_ref[0]

        barrier = pltpu.get_barrier_semaphore()
        pl.semaphore_signal(
            barrier,
            inc=1,
            device_id=(1 - my_x, my_y),
            device_id_type=pl.DeviceIdType.MESH,
        )
        pl.semaphore_wait(barrier, 1)

        rdma = pltpu.make_async_remote_copy(
            src_ref=x_ref,
            dst_ref=pbuf_ref,
            send_sem=send_sem,
            recv_sem=recv_sem,
            device_id=(1 - my_x, my_y),
            device_id_type=pl.DeviceIdType.MESH,
        )
        rdma.start()
        rdma.wait()

        rows = lax.broadcasted_iota(jnp.int32, (N, 1), 0)
        s = 1 - 2 * my_x
        own_mask = s * rows < s * z + my_x
        rolled = pltpu.roll(pbuf_ref[...], z, 0)
        out_ref[...] = jnp.where(own_mask, x_ref[...], rolled)

    return pl.pallas_call(
        body,
        out_shape=jax.ShapeDtypeStruct((N, D), jnp.float32),
        in_specs=[
            pl.BlockSpec(memory_space=pltpu.SMEM),
            pl.BlockSpec(memory_space=pltpu.VMEM),
        ],
        out_specs=pl.BlockSpec(memory_space=pltpu.VMEM),
        scratch_shapes=[
            pltpu.VMEM((N, D), jnp.float32),
            pltpu.SemaphoreType.DMA,
            pltpu.SemaphoreType.DMA,
        ],
        compiler_params=pltpu.CompilerParams(collective_id=0),
    )(z, xs)


# device time: 29010 ns/iter; 1.3772x vs baseline; 1.3772x over previous
import jax
import jax.numpy as jnp
from jax import lax
from jax.experimental import pallas as pl
from jax.experimental.pallas import tpu as pltpu

N = 1024
D = 512
C = 64
KMAX = N // C


def kernel(x, dest):
    order = jnp.argsort(dest)
    xs = jnp.take(x, order, axis=0)
    z = jnp.sum(dest == 0).astype(jnp.int32).reshape(1)

    def body(z_ref, x_ref, out_ref, pbuf_ref, send_sems, recv_sems):
        my_x = lax.axis_index("x")
        my_y = lax.axis_index("y")
        partner = (1 - my_x, my_y)
        z = z_ref[0]

        barrier = pltpu.get_barrier_semaphore()
        pl.semaphore_signal(
            barrier,
            inc=1,
            device_id=partner,
            device_id_type=pl.DeviceIdType.MESH,
        )
        pl.semaphore_wait(barrier, 1)

        nz = N - z
        base = jnp.where(my_x == 0, (z // 8) * 8, 0)
        ln = jnp.where(my_x == 0, N - base, ((z + 7) // 8) * 8)
        k = (ln + C - 1) // C
        ln_p = jnp.where(
            my_x == 0, ((nz + 7) // 8) * 8, N - (nz // 8) * 8
        )
        k_recv = (ln_p + C - 1) // C

        def chunk_desc(j, off):
            return pltpu.make_async_remote_copy(
                src_ref=x_ref.at[pl.ds(off, C)],
                dst_ref=pbuf_ref.at[pl.ds(off, C)],
                send_sem=send_sems.at[j],
                recv_sem=recv_sems.at[j],
                device_id=partner,
                device_id_type=pl.DeviceIdType.MESH,
            )

        def my_off(j):
            off = jnp.where(
                j == k - 1,
                jnp.maximum(base + ln - C, 0),
                base + j * C,
            )
            return pl.multiple_of(off, 8)

        for j in range(KMAX):

            @pl.when(j < k)
            def _():
                chunk_desc(j, my_off(j)).start()

        for j in range(KMAX):

            @pl.when(j < k_recv)
            def _():
                chunk_desc(j, pl.multiple_of(j * C, 8)).wait_recv()

        rows = lax.broadcasted_iota(jnp.int32, (N, 1), 0)
        s = 1 - 2 * my_x
        own_mask = s * rows < s * z + my_x
        rolled = pltpu.roll(pbuf_ref[...], z, 0)
        out_ref[...] = jnp.where(own_mask, x_ref[...], rolled)

        for j in range(KMAX):

            @pl.when(j < k)
            def _():
                chunk_desc(j, my_off(j)).wait_send()

    return pl.pallas_call(
        body,
        out_shape=jax.ShapeDtypeStruct((N, D), jnp.float32),
        in_specs=[
            pl.BlockSpec(memory_space=pltpu.SMEM),
            pl.BlockSpec(memory_space=pltpu.VMEM),
        ],
        out_specs=pl.BlockSpec(memory_space=pltpu.VMEM),
        scratch_shapes=[
            pltpu.VMEM((N, D), jnp.float32),
            pltpu.SemaphoreType.DMA((KMAX,)),
            pltpu.SemaphoreType.DMA((KMAX,)),
        ],
        compiler_params=pltpu.CompilerParams(collective_id=0),
    )(z, xs)


# device time: 27879 ns/iter; 1.4331x vs baseline; 1.0406x over previous
import jax
import jax.numpy as jnp
from jax import lax
from jax.experimental import pallas as pl
from jax.experimental.pallas import tpu as pltpu

N = 1024
D = 512
C = 32
KMAX = N // C


def kernel(x, dest):
    key = dest * N + lax.iota(jnp.int32, N)
    order = jnp.sort(key) & (N - 1)
    xs = jnp.take(x, order, axis=0)
    z = (N - jnp.sum(dest)).astype(jnp.int32).reshape(1)

    def body(z_ref, x_ref, out_ref, pbuf_ref, send_sems, recv_sems):
        my_x = lax.axis_index("x")
        my_y = lax.axis_index("y")
        partner = (1 - my_x, my_y)
        z = z_ref[0]

        barrier = pltpu.get_barrier_semaphore()
        pl.semaphore_signal(
            barrier,
            inc=1,
            device_id=partner,
            device_id_type=pl.DeviceIdType.MESH,
        )
        pl.semaphore_wait(barrier, 1)

        nz = N - z
        base = jnp.where(my_x == 0, (z // 8) * 8, 0)
        ln = jnp.where(my_x == 0, N - base, ((z + 7) // 8) * 8)
        k = (ln + C - 1) // C
        ln_p = jnp.where(
            my_x == 0, ((nz + 7) // 8) * 8, N - (nz // 8) * 8
        )
        k_recv = (ln_p + C - 1) // C

        def chunk_desc(j, off):
            return pltpu.make_async_remote_copy(
                src_ref=x_ref.at[pl.ds(off, C)],
                dst_ref=pbuf_ref.at[pl.ds(off, C)],
                send_sem=send_sems.at[j],
                recv_sem=recv_sems.at[j],
                device_id=partner,
                device_id_type=pl.DeviceIdType.MESH,
            )

        def my_off(j):
            off = jnp.where(
                j == k - 1,
                jnp.maximum(base + ln - C, 0),
                base + j * C,
            )
            return pl.multiple_of(off, 8)

        for j in range(KMAX):

            @pl.when(j < k)
            def _():
                chunk_desc(j, my_off(j)).start()

        for j in range(KMAX):

            @pl.when(j < k_recv)
            def _():
                chunk_desc(j, pl.multiple_of(j * C, 8)).wait_recv()

        rows = lax.broadcasted_iota(jnp.int32, (N, 1), 0)
        s = 1 - 2 * my_x
        own_mask = s * rows < s * z + my_x
        rolled = pltpu.roll(pbuf_ref[...], z, 0)
        out_ref[...] = jnp.where(own_mask, x_ref[...], rolled)

        for j in range(KMAX):

            @pl.when(j < k)
            def _():
                chunk_desc(j, my_off(j)).wait_send()

    return pl.pallas_call(
        body,
        out_shape=jax.ShapeDtypeStruct((N, D), jnp.float32),
        in_specs=[
            pl.BlockSpec(memory_space=pltpu.SMEM),
            pl.BlockSpec(memory_space=pltpu.VMEM),
        ],
        out_specs=pl.BlockSpec(memory_space=pltpu.VMEM),
        scratch_shapes=[
            pltpu.VMEM((N, D), jnp.float32),
            pltpu.SemaphoreType.DMA((KMAX,)),
            pltpu.SemaphoreType.DMA((KMAX,)),
        ],
        compiler_params=pltpu.CompilerParams(collective_id=0),
    )(z, xs)


# device time: 27827 ns/iter; 1.4358x vs baseline; 1.0019x over previous
import jax
import jax.numpy as jnp
from jax import lax
from jax.experimental import pallas as pl
from jax.experimental.pallas import tpu as pltpu

N = 1024
D = 512
C = 32
KMAX = N // C


def kernel(x, dest):
    key = dest * N + lax.iota(jnp.int32, N)
    order = jnp.sort(key) & (N - 1)
    xs = jnp.take(x, order, axis=0)
    d2 = dest.reshape(8, 128)

    def body(d_ref, x_ref, out_ref, pbuf_ref, send_sems, recv_sems):
        my_x = lax.axis_index("x")
        my_y = lax.axis_index("y")
        partner = (1 - my_x, my_y)
        z = N - jnp.sum(d_ref[...])

        barrier = pltpu.get_barrier_semaphore()
        pl.semaphore_signal(
            barrier,
            inc=1,
            device_id=partner,
            device_id_type=pl.DeviceIdType.MESH,
        )
        pl.semaphore_wait(barrier, 1)

        nz = N - z
        base = jnp.where(my_x == 0, (z // 8) * 8, 0)
        ln = jnp.where(my_x == 0, N - base, ((z + 7) // 8) * 8)
        k = (ln + C - 1) // C
        ln_p = jnp.where(
            my_x == 0, ((nz + 7) // 8) * 8, N - (nz // 8) * 8
        )
        k_recv = (ln_p + C - 1) // C

        def chunk_desc(j, off):
            return pltpu.make_async_remote_copy(
                src_ref=x_ref.at[pl.ds(off, C)],
                dst_ref=pbuf_ref.at[pl.ds(off, C)],
                send_sem=send_sems.at[j],
                recv_sem=recv_sems.at[j],
                device_id=partner,
                device_id_type=pl.DeviceIdType.MESH,
            )

        def my_off(j):
            off = jnp.where(
                j == k - 1,
                jnp.maximum(base + ln - C, 0),
                base + j * C,
            )
            return pl.multiple_of(off, 8)

        for j in range(KMAX):

            @pl.when(j < k)
            def _():
                chunk_desc(j, my_off(j)).start()

        for j in range(KMAX):

            @pl.when(j < k_recv)
            def _():
                chunk_desc(j, pl.multiple_of(j * C, 8)).wait_recv()

        rows = lax.broadcasted_iota(jnp.int32, (N, 1), 0)
        s = 1 - 2 * my_x
        own_mask = s * rows < s * z + my_x
        rolled = pltpu.roll(pbuf_ref[...], z, 0)
        out_ref[...] = jnp.where(own_mask, x_ref[...], rolled)

        for j in range(KMAX):

            @pl.when(j < k)
            def _():
                chunk_desc(j, my_off(j)).wait_send()

    return pl.pallas_call(
        body,
        out_shape=jax.ShapeDtypeStruct((N, D), jnp.float32),
        in_specs=[
            pl.BlockSpec(memory_space=pltpu.VMEM),
            pl.BlockSpec(memory_space=pltpu.VMEM),
        ],
        out_specs=pl.BlockSpec(memory_space=pltpu.VMEM),
        scratch_shapes=[
            pltpu.VMEM((N, D), jnp.float32),
            pltpu.SemaphoreType.DMA((KMAX,)),
            pltpu.SemaphoreType.DMA((KMAX,)),
        ],
        compiler_params=pltpu.CompilerParams(collective_id=0),
    )(d2, xs)


# device time: 25762 ns/iter; 1.5509x vs baseline; 1.0802x over previous
import jax
import jax.numpy as jnp
from jax import lax
from jax.experimental import pallas as pl
from jax.experimental.pallas import tpu as pltpu

N = 1024
D = 512
C = 32
KMAX = N // C


def kernel(x, dest):
    key = dest * N + lax.iota(jnp.int32, N)
    order = jnp.sort(key) & (N - 1)
    xs = jnp.take(x, order, axis=0)
    d2 = dest.reshape(8, 128)

    def body(d_ref, x_ref, out_ref, pbuf_ref, send_sems, recv_sems, fwd_sems):
        my_x = lax.axis_index("x")
        my_y = lax.axis_index("y")
        xp = (1 - my_x, my_y)
        yp = (my_x, 1 - my_y)
        z = N - jnp.sum(d_ref[...])

        barrier = pltpu.get_barrier_semaphore()
        for nbr in (xp, yp):
            pl.semaphore_signal(
                barrier,
                inc=1,
                device_id=nbr,
                device_id_type=pl.DeviceIdType.MESH,
            )
        pl.semaphore_wait(barrier, 2)

        nz = N - z
        base = jnp.where(my_x == 0, (z // 8) * 8, 0)
        ln = jnp.where(my_x == 0, N - base, ((z + 7) // 8) * 8)
        k = (ln + C - 1) // C
        base_p = jnp.where(my_x == 0, 0, (nz // 8) * 8)
        ln_p = jnp.where(my_x == 0, ((nz + 7) // 8) * 8, N - base_p)
        k_recv = (ln_p + C - 1) // C

        def chunk_off(j, b, l, kk):
            off = jnp.where(
                j == kk - 1, jnp.maximum(b + l - C, 0), b + j * C
            )
            return pl.multiple_of(off, 8)

        def my_off(j):
            return chunk_off(j, base, ln, k)

        def p_off(j):
            return chunk_off(j, base_p, ln_p, k_recv)

        def rdma(src_off, dst_off, ssem, rsem, dev):
            return pltpu.make_async_remote_copy(
                src_ref=x_ref.at[pl.ds(src_off, C)],
                dst_ref=pbuf_ref.at[pl.ds(dst_off, C)],
                send_sem=ssem,
                recv_sem=rsem,
                device_id=dev,
                device_id_type=pl.DeviceIdType.MESH,
            )

        def fwd_rdma(j):
            off = p_off(j)
            return pltpu.make_async_remote_copy(
                src_ref=pbuf_ref.at[pl.ds(off, C)],
                dst_ref=pbuf_ref.at[pl.ds(off, C)],
                send_sem=fwd_sems.at[j],
                recv_sem=recv_sems.at[j],
                device_id=yp,
                device_id_type=pl.DeviceIdType.MESH,
            )

        for j in range(KMAX):

            @pl.when((j < k) & (j % 2 == my_y))
            def _():
                o = my_off(j)
                rdma(o, o, send_sems.at[j], recv_sems.at[j], xp).start()

        for j in range(KMAX):

            @pl.when((j < k_recv) & (j % 2 == my_y))
            def _():
                o = p_off(j)
                rdma(o, o, send_sems.at[j], recv_sems.at[j], xp).wait_recv()
                fwd_rdma(j).start()

        for j in range(KMAX):

            @pl.when((j < k_recv) & (j % 2 != my_y))
            def _():
                o = p_off(j)
                rdma(o, o, send_sems.at[j], recv_sems.at[j], yp).wait_recv()

        rows = lax.broadcasted_iota(jnp.int32, (N, 1), 0)
        s = 1 - 2 * my_x
        own_mask = s * rows < s * z + my_x
        rolled = pltpu.roll(pbuf_ref[...], z, 0)
        out_ref[...] = jnp.where(own_mask, x_ref[...], rolled)

        for j in range(KMAX):

            @pl.when((j < k) & (j % 2 == my_y))
            def _():
                o = my_off(j)
                rdma(o, o, send_sems.at[j], recv_sems.at[j], xp).wait_send()

            @pl.when((j < k_recv) & (j % 2 == my_y))
            def _():
                fwd_rdma(j).wait_send()

    return pl.pallas_call(
        body,
        out_shape=jax.ShapeDtypeStruct((N, D), jnp.float32),
        in_specs=[
            pl.BlockSpec(memory_space=pltpu.VMEM),
            pl.BlockSpec(memory_space=pltpu.VMEM),
        ],
        out_specs=pl.BlockSpec(memory_space=pltpu.VMEM),
        scratch_shapes=[
            pltpu.VMEM((N, D), jnp.float32),
            pltpu.SemaphoreType.DMA((KMAX,)),
            pltpu.SemaphoreType.DMA((KMAX,)),
            pltpu.SemaphoreType.DMA((KMAX,)),
        ],
        compiler_params=pltpu.CompilerParams(collective_id=0),
    )(d2, xs)


# device time: 25540 ns/iter; 1.5644x vs baseline; 1.0087x over previous
import jax
import jax.numpy as jnp
from jax import lax
from jax.experimental import pallas as pl
from jax.experimental.pallas import tpu as pltpu

N = 1024
D = 512
C = 32
KMAX = N // C


def kernel(x, dest):
    key = (dest * N + lax.iota(jnp.int32, N)).astype(jnp.int16)
    order = jnp.sort(key).astype(jnp.int32) & (N - 1)
    xs = jnp.take(x, order, axis=0)
    d2 = dest.reshape(8, 128)

    def body(d_ref, x_ref, out_ref, pbuf_ref, send_sems, recv_sems, fwd_sems):
        my_x = lax.axis_index("x")
        my_y = lax.axis_index("y")
        xp = (1 - my_x, my_y)
        yp = (my_x, 1 - my_y)
        z = N - jnp.sum(d_ref[...])

        barrier = pltpu.get_barrier_semaphore()
        for nbr in (xp, yp):
            pl.semaphore_signal(
                barrier,
                inc=1,
                device_id=nbr,
                device_id_type=pl.DeviceIdType.MESH,
            )
        pl.semaphore_wait(barrier, 2)

        nz = N - z
        base = jnp.where(my_x == 0, (z // 8) * 8, 0)
        ln = jnp.where(my_x == 0, N - base, ((z + 7) // 8) * 8)
        k = (ln + C - 1) // C
        base_p = jnp.where(my_x == 0, 0, (nz // 8) * 8)
        ln_p = jnp.where(my_x == 0, ((nz + 7) // 8) * 8, N - base_p)
        k_recv = (ln_p + C - 1) // C

        def chunk_off(j, b, l, kk):
            off = jnp.where(
                j == kk - 1, jnp.maximum(b + l - C, 0), b + j * C
            )
            return pl.multiple_of(off, 8)

        def my_off(j):
            return chunk_off(j, base, ln, k)

        def p_off(j):
            return chunk_off(j, base_p, ln_p, k_recv)

        def rdma(src_off, dst_off, ssem, rsem, dev):
            return pltpu.make_async_remote_copy(
                src_ref=x_ref.at[pl.ds(src_off, C)],
                dst_ref=pbuf_ref.at[pl.ds(dst_off, C)],
                send_sem=ssem,
                recv_sem=rsem,
                device_id=dev,
                device_id_type=pl.DeviceIdType.MESH,
            )

        def fwd_rdma(j):
            off = p_off(j)
            return pltpu.make_async_remote_copy(
                src_ref=pbuf_ref.at[pl.ds(off, C)],
                dst_ref=pbuf_ref.at[pl.ds(off, C)],
                send_sem=fwd_sems.at[j],
                recv_sem=recv_sems.at[j],
                device_id=yp,
                device_id_type=pl.DeviceIdType.MESH,
            )

        for j in range(KMAX):

            @pl.when((j < k) & (j % 2 == my_y))
            def _():
                o = my_off(j)
                rdma(o, o, send_sems.at[j], recv_sems.at[j], xp).start()

        for j in range(KMAX):

            @pl.when((j < k_recv) & (j % 2 == my_y))
            def _():
                o = p_off(j)
                rdma(o, o, send_sems.at[j], recv_sems.at[j], xp).wait_recv()
                fwd_rdma(j).start()

        for j in range(KMAX):

            @pl.when((j < k_recv) & (j % 2 != my_y))
            def _():
                o = p_off(j)
                rdma(o, o, send_sems.at[j], recv_sems.at[j], yp).wait_recv()

        rows = lax.broadcasted_iota(jnp.int32, (N, 1), 0)
        s = 1 - 2 * my_x
        own_mask = s * rows < s * z + my_x
        rolled = pltpu.roll(pbuf_ref[...], z, 0)
        out_ref[...] = jnp.where(own_mask, x_ref[...], rolled)

        for j in range(KMAX):

            @pl.when((j < k) & (j % 2 == my_y))
            def _():
                o = my_off(j)
                rdma(o, o, send_sems.at[j], recv_sems.at[j], xp).wait_send()

            @pl.when((j < k_recv) & (j % 2 == my_y))
            def _():
                fwd_rdma(j).wait_send()

    return pl.pallas_call(
        body,
        out_shape=jax.ShapeDtypeStruct((N, D), jnp.float32),
        in_specs=[
            pl.BlockSpec(memory_space=pltpu.VMEM),
            pl.BlockSpec(memory_space=pltpu.VMEM),
        ],
        out_specs=pl.BlockSpec(memory_space=pltpu.VMEM),
        scratch_shapes=[
            pltpu.VMEM((N, D), jnp.float32),
            pltpu.SemaphoreType.DMA((KMAX,)),
            pltpu.SemaphoreType.DMA((KMAX,)),
            pltpu.SemaphoreType.DMA((KMAX,)),
        ],
        compiler_params=pltpu.CompilerParams(collective_id=0),
    )(d2, xs)
